# Initial kernel scaffold; baseline (speedup 1.0000x reference)
import jax
import jax.numpy as jnp
from jax.experimental import pallas as pl


def kernel(pred_verts, pred_faces, gt_verts, gt_faces):
    raise NotImplementedError("write your pallas kernel here")


# speedup vs baseline: 78.2407x; 78.2407x over previous
import functools

import jax
import jax.numpy as jnp
from jax import lax
from jax.experimental import pallas as pl
from jax.experimental.pallas import tpu as pltpu
from jax.experimental.pallas import tpu_sc as plsc

NC = 2
NS = 16
L = 16
CHUNK = 128


def _rsqrt(x):
    i = plsc.bitcast(x, jnp.int32)
    i = jnp.int32(0x5F3759DF) - lax.shift_right_logical(i, 1)
    y = plsc.bitcast(i, jnp.float32)
    for _ in range(3):
        y = y * (1.5 - 0.5 * x * y * y)
    return y


def _col(ref, rows, c):
    return plsc.load_gather(ref, [rows, jnp.full((L,), c, jnp.int32)])


def _sc_kernel_body(V, V8, NCH, VPT,
                    pv_hbm, gv_hbm, pfg_hbm, pfs_hbm, gfg_hbm, zero_hbm,
                    out_hbm,
                    acc, pgi, psi, ggi, p0r, p1r, p2r, g0r, g1r, g2r,
                    srows, accv, vrtv, psum):
    c = lax.axis_index("c")
    s = lax.axis_index("s")
    iota16 = lax.iota(jnp.int32, L)
    ones16 = jnp.full((L,), 1.0, jnp.float32)
    zeros16 = jnp.zeros((L,), jnp.float32)

    psum[0, :] = zeros16
    psum[1, :] = zeros16

    def normal(x0, y0, z0, x1, y1, z1, x2, y2, z2):
        e1x, e1y, e1z = x1 - x0, y1 - y0, z1 - z0
        e2x, e2y, e2z = x2 - x0, y2 - y0, z2 - z0
        nx = e1y * e2z - e1z * e2y
        ny = e1z * e2x - e1x * e2z
        nz = e1x * e2y - e1y * e2x
        ss = nx * nx + ny * ny + nz * nz
        r = jnp.where(ss <= 1e-24, 1e12, _rsqrt(ss))
        return nx * r, ny * r, nz * r

    def batch_body(sb, carry):
        b = c * 4 + sb
        pltpu.sync_copy(zero_hbm, acc.at[pl.ds(s * VPT, VPT), :])
        for j in range(3):
            pltpu.sync_copy(pfg_hbm.at[b, j, s], pgi.at[pl.ds(j * NCH, NCH)])
            pltpu.sync_copy(pfs_hbm.at[b, j, s], psi.at[pl.ds(j * NCH, NCH)])
            pltpu.sync_copy(gfg_hbm.at[b, j, s], ggi.at[pl.ds(j * NCH, NCH)])
        plsc.subcore_barrier()

        def chunk_body(ck, carry2):
            pltpu.sync_copy(pv_hbm.at[pgi.at[ck]], p0r)
            pltpu.sync_copy(pv_hbm.at[pgi.at[NCH + ck]], p1r)
            pltpu.sync_copy(pv_hbm.at[pgi.at[2 * NCH + ck]], p2r)
            pltpu.sync_copy(gv_hbm.at[ggi.at[ck]], g0r)
            pltpu.sync_copy(gv_hbm.at[ggi.at[NCH + ck]], g1r)
            pltpu.sync_copy(gv_hbm.at[ggi.at[2 * NCH + ck]], g2r)
            gacc = zeros16
            for j8 in range(CHUNK // L):
                rows = j8 * L + iota16
                x0, y0, z0 = _col(p0r, rows, 0), _col(p0r, rows, 1), _col(p0r, rows, 2)
                x1, y1, z1 = _col(p1r, rows, 0), _col(p1r, rows, 1), _col(p1r, rows, 2)
                x2, y2, z2 = _col(p2r, rows, 0), _col(p2r, rows, 1), _col(p2r, rows, 2)
                plsc.store_scatter(srows, [rows, jnp.full((L,), 0, jnp.int32)], x0 + x1 + x2)
                plsc.store_scatter(srows, [rows, jnp.full((L,), 1, jnp.int32)], y0 + y1 + y2)
                plsc.store_scatter(srows, [rows, jnp.full((L,), 2, jnp.int32)], z0 + z1 + z2)
                plsc.store_scatter(srows, [rows, jnp.full((L,), 3, jnp.int32)], ones16)
                pnx, pny, pnz = normal(x0, y0, z0, x1, y1, z1, x2, y2, z2)
                a0, b0, c0 = _col(g0r, rows, 0), _col(g0r, rows, 1), _col(g0r, rows, 2)
                a1, b1, c1 = _col(g1r, rows, 0), _col(g1r, rows, 1), _col(g1r, rows, 2)
                a2, b2, c2 = _col(g2r, rows, 0), _col(g2r, rows, 1), _col(g2r, rows, 2)
                gnx, gny, gnz = normal(a0, b0, c0, a1, b1, c1, a2, b2, c2)
                dx, dy, dz = pnx - gnx, pny - gny, pnz - gnz
                gacc = gacc + dx * dx + dy * dy + dz * dz
            plsc.addupdate(psum.at[1], gacc)
            pltpu.sync_copy(srows, acc.at[psi.at[ck]], add=True)
            pltpu.sync_copy(srows, acc.at[psi.at[NCH + ck]], add=True)
            pltpu.sync_copy(srows, acc.at[psi.at[2 * NCH + ck]], add=True)
            return carry2

        lax.fori_loop(0, NCH, chunk_body, 0)
        plsc.subcore_barrier()

        HVPT = VPT // 2

        def half_body(h, carry2):
            base = s * VPT + h * HVPT
            pltpu.sync_copy(acc.at[pl.ds(base, HVPT), :], accv)
            pltpu.sync_copy(pv_hbm.at[pl.ds(b * V8 + base, HVPT), :], vrtv)
            vid0 = base

            def row_body(i, carry3):
                rows = i * L + iota16
                ax, ay, az = _col(accv, rows, 0), _col(accv, rows, 1), _col(accv, rows, 2)
                an = _col(accv, rows, 3)
                vx, vy, vz = _col(vrtv, rows, 0), _col(vrtv, rows, 1), _col(vrtv, rows, 2)
                inv = 1.0 / jnp.maximum(an + an, 1.0)
                lx = (ax - an * vx) * inv - vx
                ly = (ay - an * vy) * inv - vy
                lz = (az - an * vz) * inv - vz
                ss = lx * lx + ly * ly + lz * lz + 1e-12
                nrm = ss * _rsqrt(ss)
                nrm = jnp.where(vid0 + rows < V, nrm, 0.0)
                plsc.addupdate(psum.at[0], nrm)
                return carry3

            return lax.fori_loop(0, HVPT // L, row_body, carry2)

        lax.fori_loop(0, 2, half_body, 0)
        plsc.subcore_barrier()
        return carry

    lax.fori_loop(0, 4, batch_body, 0)
    pltpu.sync_copy(psum, out_hbm.at[c, s])


def kernel(pred_verts, pred_faces, gt_verts, gt_faces):
    B, V, _ = pred_verts.shape
    F = pred_faces.shape[1]
    VPT = ((V + L * NS - 1) // (L * NS)) * L
    V8 = VPT * NS
    FPT = ((F + CHUNK * NS - 1) // (CHUNK * NS)) * CHUNK
    FP = FPT * NS
    NCH = FPT // CHUNK

    f32 = jnp.float32
    pv8 = jnp.pad(pred_verts, ((0, 0), (0, V8 - V), (0, 5))).reshape(B * V8, 8)
    gv8 = jnp.pad(gt_verts, ((0, 0), (0, V8 - V), (0, 5))).reshape(B * V8, 8)
    boff = (jnp.arange(B, dtype=jnp.int32) * V8)[:, None, None]

    def blocks(faces, off):
        fp = jnp.pad(faces.astype(jnp.int32), ((0, 0), (0, FP - F), (0, 0)),
                     constant_values=V) + off
        return jnp.transpose(fp, (0, 2, 1)).reshape(B, 3, NS, NCH, CHUNK)

    pfg = blocks(pred_faces, boff)
    pfs = blocks(pred_faces, 0)
    gfg = blocks(gt_faces, boff)
    zero_rows = jnp.zeros((VPT, 8), f32)

    mesh = plsc.VectorSubcoreMesh(core_axis_name="c", subcore_axis_name="s",
                                  num_cores=NC, num_subcores=NS)
    body = functools.partial(_sc_kernel_body, V, V8, NCH, VPT)
    out = pl.kernel(
        body,
        out_type=jax.ShapeDtypeStruct((NC, NS, 2, L), f32),
        mesh=mesh,
        compiler_params=pltpu.CompilerParams(
            needs_layout_passes=False, use_tc_tiling_on_sc=False),
        scratch_types=[
            pltpu.VMEM_SHARED((V8, 8), f32),
            pltpu.VMEM((3 * NCH, CHUNK), jnp.int32),
            pltpu.VMEM((3 * NCH, CHUNK), jnp.int32),
            pltpu.VMEM((3 * NCH, CHUNK), jnp.int32),
            pltpu.VMEM((CHUNK, 8), f32),
            pltpu.VMEM((CHUNK, 8), f32),
            pltpu.VMEM((CHUNK, 8), f32),
            pltpu.VMEM((CHUNK, 8), f32),
            pltpu.VMEM((CHUNK, 8), f32),
            pltpu.VMEM((CHUNK, 8), f32),
            pltpu.VMEM((CHUNK, 8), f32),
            pltpu.VMEM((VPT // 2, 8), f32),
            pltpu.VMEM((VPT // 2, 8), f32),
            pltpu.VMEM((2, L), f32),
        ],
    )(pv8, gv8, pfg, pfs, gfg, zero_rows)

    lap_sum = jnp.sum(out[:, :, 0, :])
    grad_sum = jnp.sum(out[:, :, 1, :])
    return 0.1 * lap_sum / (B * V) + 0.05 * grad_sum / (B * F * 3)


# speedup vs baseline: 111.2151x; 1.4214x over previous
import functools

import jax
import jax.numpy as jnp
from jax import lax
from jax.experimental import pallas as pl
from jax.experimental.pallas import tpu as pltpu
from jax.experimental.pallas import tpu_sc as plsc

NC = 2
NS = 16
L = 16
CHUNK = 128


def _rsqrt(x):
    i = plsc.bitcast(x, jnp.int32)
    i = jnp.int32(0x5F3759DF) - lax.shift_right_logical(i, 1)
    y = plsc.bitcast(i, jnp.float32)
    for _ in range(3):
        y = y * (1.5 - 0.5 * x * y * y)
    return y


def _col(ref, rows, c):
    return plsc.load_gather(ref, [rows, jnp.full((L,), c, jnp.int32)])


def _sc_kernel_body(V, V8, NCH, VPT,
                    pv_hbm, gv_hbm, pfg_hbm, pfs_hbm, gfg_hbm, zero_hbm,
                    out_hbm,
                    acc, pgi, psi, ggi, p0r, p1r, p2r, g0r, g1r, g2r,
                    srows, accv, vrtv, psum, sem):
    c = lax.axis_index("c")
    s = lax.axis_index("s")
    iota16 = lax.iota(jnp.int32, L)
    ones16 = jnp.full((L,), 1.0, jnp.float32)
    zeros16 = jnp.zeros((L,), jnp.float32)

    psum[0, :] = zeros16
    psum[1, :] = zeros16

    def normal(x0, y0, z0, x1, y1, z1, x2, y2, z2):
        e1x, e1y, e1z = x1 - x0, y1 - y0, z1 - z0
        e2x, e2y, e2z = x2 - x0, y2 - y0, z2 - z0
        nx = e1y * e2z - e1z * e2y
        ny = e1z * e2x - e1x * e2z
        nz = e1x * e2y - e1y * e2x
        ss = nx * nx + ny * ny + nz * nz
        r = jnp.where(ss <= 1e-24, 1e12, _rsqrt(ss))
        return nx * r, ny * r, nz * r

    def batch_body(sb, carry):
        b = c * 4 + sb
        pltpu.sync_copy(zero_hbm, acc.at[pl.ds(s * VPT, VPT), :])
        for j in range(3):
            pltpu.sync_copy(pfg_hbm.at[b, j, s], pgi.at[pl.ds(j * NCH, NCH)])
            pltpu.sync_copy(pfs_hbm.at[b, j, s], psi.at[pl.ds(j * NCH, NCH)])
            pltpu.sync_copy(gfg_hbm.at[b, j, s], ggi.at[pl.ds(j * NCH, NCH)])
        plsc.subcore_barrier()

        def chunk_body(ck, carry2):
            ds = [pltpu.async_copy(pv_hbm.at[pgi.at[ck]], p0r, sem),
                  pltpu.async_copy(pv_hbm.at[pgi.at[NCH + ck]], p1r, sem),
                  pltpu.async_copy(pv_hbm.at[pgi.at[2 * NCH + ck]], p2r, sem),
                  pltpu.async_copy(gv_hbm.at[ggi.at[ck]], g0r, sem),
                  pltpu.async_copy(gv_hbm.at[ggi.at[NCH + ck]], g1r, sem),
                  pltpu.async_copy(gv_hbm.at[ggi.at[2 * NCH + ck]], g2r, sem)]
            for d in ds:
                d.wait()
            gacc = zeros16
            for j8 in range(CHUNK // L):
                rows = j8 * L + iota16
                x0, y0, z0 = _col(p0r, rows, 0), _col(p0r, rows, 1), _col(p0r, rows, 2)
                x1, y1, z1 = _col(p1r, rows, 0), _col(p1r, rows, 1), _col(p1r, rows, 2)
                x2, y2, z2 = _col(p2r, rows, 0), _col(p2r, rows, 1), _col(p2r, rows, 2)
                plsc.store_scatter(srows, [rows, jnp.full((L,), 0, jnp.int32)], x0 + x1 + x2)
                plsc.store_scatter(srows, [rows, jnp.full((L,), 1, jnp.int32)], y0 + y1 + y2)
                plsc.store_scatter(srows, [rows, jnp.full((L,), 2, jnp.int32)], z0 + z1 + z2)
                plsc.store_scatter(srows, [rows, jnp.full((L,), 3, jnp.int32)], ones16)
                pnx, pny, pnz = normal(x0, y0, z0, x1, y1, z1, x2, y2, z2)
                a0, b0, c0 = _col(g0r, rows, 0), _col(g0r, rows, 1), _col(g0r, rows, 2)
                a1, b1, c1 = _col(g1r, rows, 0), _col(g1r, rows, 1), _col(g1r, rows, 2)
                a2, b2, c2 = _col(g2r, rows, 0), _col(g2r, rows, 1), _col(g2r, rows, 2)
                gnx, gny, gnz = normal(a0, b0, c0, a1, b1, c1, a2, b2, c2)
                dx, dy, dz = pnx - gnx, pny - gny, pnz - gnz
                gacc = gacc + dx * dx + dy * dy + dz * dz
            plsc.addupdate(psum.at[1], gacc)
            pltpu.sync_copy(srows, acc.at[psi.at[ck]], add=True)
            pltpu.sync_copy(srows, acc.at[psi.at[NCH + ck]], add=True)
            pltpu.sync_copy(srows, acc.at[psi.at[2 * NCH + ck]], add=True)
            return carry2

        lax.fori_loop(0, NCH, chunk_body, 0)
        plsc.subcore_barrier()

        HVPT = VPT // 2

        def half_body(h, carry2):
            base = s * VPT + h * HVPT
            pltpu.sync_copy(acc.at[pl.ds(base, HVPT), :], accv)
            pltpu.sync_copy(pv_hbm.at[pl.ds(b * V8 + base, HVPT), :], vrtv)
            vid0 = base

            def row_body(i, carry3):
                rows = i * L + iota16
                ax, ay, az = _col(accv, rows, 0), _col(accv, rows, 1), _col(accv, rows, 2)
                an = _col(accv, rows, 3)
                vx, vy, vz = _col(vrtv, rows, 0), _col(vrtv, rows, 1), _col(vrtv, rows, 2)
                inv = 1.0 / jnp.maximum(an + an, 1.0)
                lx = (ax - an * vx) * inv - vx
                ly = (ay - an * vy) * inv - vy
                lz = (az - an * vz) * inv - vz
                ss = lx * lx + ly * ly + lz * lz + 1e-12
                nrm = ss * _rsqrt(ss)
                nrm = jnp.where(vid0 + rows < V, nrm, 0.0)
                plsc.addupdate(psum.at[0], nrm)
                return carry3

            return lax.fori_loop(0, HVPT // L, row_body, carry2)

        lax.fori_loop(0, 2, half_body, 0)
        plsc.subcore_barrier()
        return carry

    lax.fori_loop(0, 4, batch_body, 0)
    pltpu.sync_copy(psum, out_hbm.at[c, s])


def kernel(pred_verts, pred_faces, gt_verts, gt_faces):
    B, V, _ = pred_verts.shape
    F = pred_faces.shape[1]
    VPT = ((V + L * NS - 1) // (L * NS)) * L
    V8 = VPT * NS
    FPT = ((F + CHUNK * NS - 1) // (CHUNK * NS)) * CHUNK
    FP = FPT * NS
    NCH = FPT // CHUNK

    f32 = jnp.float32
    pv8 = jnp.pad(pred_verts, ((0, 0), (0, V8 - V), (0, 5))).reshape(B * V8, 8)
    gv8 = jnp.pad(gt_verts, ((0, 0), (0, V8 - V), (0, 5))).reshape(B * V8, 8)
    boff = (jnp.arange(B, dtype=jnp.int32) * V8)[:, None, None]

    def blocks(faces, off):
        fp = jnp.pad(faces.astype(jnp.int32), ((0, 0), (0, FP - F), (0, 0)),
                     constant_values=V) + off
        return jnp.transpose(fp, (0, 2, 1)).reshape(B, 3, NS, NCH, CHUNK)

    pfg = blocks(pred_faces, boff)
    pfs = blocks(pred_faces, 0)
    gfg = blocks(gt_faces, boff)
    zero_rows = jnp.zeros((VPT, 8), f32)

    mesh = plsc.VectorSubcoreMesh(core_axis_name="c", subcore_axis_name="s",
                                  num_cores=NC, num_subcores=NS)
    body = functools.partial(_sc_kernel_body, V, V8, NCH, VPT)
    out = pl.kernel(
        body,
        out_type=jax.ShapeDtypeStruct((NC, NS, 2, L), f32),
        mesh=mesh,
        compiler_params=pltpu.CompilerParams(
            needs_layout_passes=False, use_tc_tiling_on_sc=False),
        scratch_types=[
            pltpu.VMEM_SHARED((V8, 8), f32),
            pltpu.VMEM((3 * NCH, CHUNK), jnp.int32),
            pltpu.VMEM((3 * NCH, CHUNK), jnp.int32),
            pltpu.VMEM((3 * NCH, CHUNK), jnp.int32),
            pltpu.VMEM((CHUNK, 8), f32),
            pltpu.VMEM((CHUNK, 8), f32),
            pltpu.VMEM((CHUNK, 8), f32),
            pltpu.VMEM((CHUNK, 8), f32),
            pltpu.VMEM((CHUNK, 8), f32),
            pltpu.VMEM((CHUNK, 8), f32),
            pltpu.VMEM((CHUNK, 8), f32),
            pltpu.VMEM((VPT // 2, 8), f32),
            pltpu.VMEM((VPT // 2, 8), f32),
            pltpu.VMEM((2, L), f32),
            pltpu.SemaphoreType.DMA,
        ],
    )(pv8, gv8, pfg, pfs, gfg, zero_rows)

    lap_sum = jnp.sum(out[:, :, 0, :])
    grad_sum = jnp.sum(out[:, :, 1, :])
    return 0.1 * lap_sum / (B * V) + 0.05 * grad_sum / (B * F * 3)


# speedup vs baseline: 115.9259x; 1.0424x over previous
import functools

import jax
import jax.numpy as jnp
from jax import lax
from jax.experimental import pallas as pl
from jax.experimental.pallas import tpu as pltpu
from jax.experimental.pallas import tpu_sc as plsc

NC = 2
NS = 16
L = 16
CHUNK = 128


def _rsqrt(x):
    i = plsc.bitcast(x, jnp.int32)
    i = jnp.int32(0x5F3759DF) - lax.shift_right_logical(i, 1)
    y = plsc.bitcast(i, jnp.float32)
    for _ in range(3):
        y = y * (1.5 - 0.5 * x * y * y)
    return y


def _col(ref, rows, c):
    return plsc.load_gather(ref, [rows, jnp.full((L,), c, jnp.int32)])


def _sc_kernel_body(V, V8, NCH, VPT,
                    pv_hbm, gv_hbm, pfg_hbm, pfs_hbm, gfg_hbm, zero_hbm,
                    out_hbm,
                    acc, pgi, psi, ggi, p0r, p1r, p2r, g0r, g1r, g2r,
                    srows, accv, vrtv, psum, sem, ssem):
    c = lax.axis_index("c")
    s = lax.axis_index("s")
    iota16 = lax.iota(jnp.int32, L)
    ones16 = jnp.full((L,), 1.0, jnp.float32)
    zeros16 = jnp.zeros((L,), jnp.float32)

    psum[0, :] = zeros16
    psum[1, :] = zeros16

    def normal(x0, y0, z0, x1, y1, z1, x2, y2, z2):
        e1x, e1y, e1z = x1 - x0, y1 - y0, z1 - z0
        e2x, e2y, e2z = x2 - x0, y2 - y0, z2 - z0
        nx = e1y * e2z - e1z * e2y
        ny = e1z * e2x - e1x * e2z
        nz = e1x * e2y - e1y * e2x
        ss = nx * nx + ny * ny + nz * nz
        r = jnp.where(ss <= 1e-24, 1e12, _rsqrt(ss))
        return nx * r, ny * r, nz * r

    def batch_body(sb, carry):
        b = c * 4 + sb
        pltpu.sync_copy(zero_hbm, acc.at[pl.ds(s * VPT, VPT), :])
        for j in range(3):
            pltpu.sync_copy(pfg_hbm.at[b, j, s], pgi.at[pl.ds(j * NCH, NCH)])
            pltpu.sync_copy(pfs_hbm.at[b, j, s], psi.at[pl.ds(j * NCH, NCH)])
            pltpu.sync_copy(gfg_hbm.at[b, j, s], ggi.at[pl.ds(j * NCH, NCH)])
        plsc.subcore_barrier()

        def chunk_body(ck, carry2):
            ds = [pltpu.async_copy(pv_hbm.at[pgi.at[ck]], p0r, sem),
                  pltpu.async_copy(pv_hbm.at[pgi.at[NCH + ck]], p1r, sem),
                  pltpu.async_copy(pv_hbm.at[pgi.at[2 * NCH + ck]], p2r, sem),
                  pltpu.async_copy(gv_hbm.at[ggi.at[ck]], g0r, sem),
                  pltpu.async_copy(gv_hbm.at[ggi.at[NCH + ck]], g1r, sem),
                  pltpu.async_copy(gv_hbm.at[ggi.at[2 * NCH + ck]], g2r, sem)]
            @pl.when(ck > 0)
            def _():
                for _ in range(3):
                    pltpu.make_async_copy(
                        zero_hbm.at[pl.ds(0, CHUNK), :], srows, ssem).wait()
            for d in ds:
                d.wait()
            gacc = zeros16
            for j8 in range(CHUNK // L):
                rows = j8 * L + iota16
                x0, y0, z0 = _col(p0r, rows, 0), _col(p0r, rows, 1), _col(p0r, rows, 2)
                x1, y1, z1 = _col(p1r, rows, 0), _col(p1r, rows, 1), _col(p1r, rows, 2)
                x2, y2, z2 = _col(p2r, rows, 0), _col(p2r, rows, 1), _col(p2r, rows, 2)
                plsc.store_scatter(srows, [rows, jnp.full((L,), 0, jnp.int32)], x0 + x1 + x2)
                plsc.store_scatter(srows, [rows, jnp.full((L,), 1, jnp.int32)], y0 + y1 + y2)
                plsc.store_scatter(srows, [rows, jnp.full((L,), 2, jnp.int32)], z0 + z1 + z2)
                plsc.store_scatter(srows, [rows, jnp.full((L,), 3, jnp.int32)], ones16)
                pnx, pny, pnz = normal(x0, y0, z0, x1, y1, z1, x2, y2, z2)
                a0, b0, c0 = _col(g0r, rows, 0), _col(g0r, rows, 1), _col(g0r, rows, 2)
                a1, b1, c1 = _col(g1r, rows, 0), _col(g1r, rows, 1), _col(g1r, rows, 2)
                a2, b2, c2 = _col(g2r, rows, 0), _col(g2r, rows, 1), _col(g2r, rows, 2)
                gnx, gny, gnz = normal(a0, b0, c0, a1, b1, c1, a2, b2, c2)
                dx, dy, dz = pnx - gnx, pny - gny, pnz - gnz
                gacc = gacc + dx * dx + dy * dy + dz * dz
            plsc.addupdate(psum.at[1], gacc)
            pltpu.async_copy(srows, acc.at[psi.at[ck]], ssem, add=True)
            pltpu.async_copy(srows, acc.at[psi.at[NCH + ck]], ssem, add=True)
            pltpu.async_copy(srows, acc.at[psi.at[2 * NCH + ck]], ssem, add=True)
            return carry2

        lax.fori_loop(0, NCH, chunk_body, 0)
        for _ in range(3):
            pltpu.make_async_copy(zero_hbm.at[pl.ds(0, CHUNK), :], srows, ssem).wait()
        plsc.subcore_barrier()

        HVPT = VPT // 2

        def half_body(h, carry2):
            base = s * VPT + h * HVPT
            pltpu.sync_copy(acc.at[pl.ds(base, HVPT), :], accv)
            pltpu.sync_copy(pv_hbm.at[pl.ds(b * V8 + base, HVPT), :], vrtv)
            vid0 = base

            def row_body(i, carry3):
                rows = i * L + iota16
                ax, ay, az = _col(accv, rows, 0), _col(accv, rows, 1), _col(accv, rows, 2)
                an = _col(accv, rows, 3)
                vx, vy, vz = _col(vrtv, rows, 0), _col(vrtv, rows, 1), _col(vrtv, rows, 2)
                inv = 1.0 / jnp.maximum(an + an, 1.0)
                lx = (ax - an * vx) * inv - vx
                ly = (ay - an * vy) * inv - vy
                lz = (az - an * vz) * inv - vz
                ss = lx * lx + ly * ly + lz * lz + 1e-12
                nrm = ss * _rsqrt(ss)
                nrm = jnp.where(vid0 + rows < V, nrm, 0.0)
                plsc.addupdate(psum.at[0], nrm)
                return carry3

            return lax.fori_loop(0, HVPT // L, row_body, carry2)

        lax.fori_loop(0, 2, half_body, 0)
        plsc.subcore_barrier()
        return carry

    lax.fori_loop(0, 4, batch_body, 0)
    pltpu.sync_copy(psum, out_hbm.at[c, s])


def kernel(pred_verts, pred_faces, gt_verts, gt_faces):
    B, V, _ = pred_verts.shape
    F = pred_faces.shape[1]
    VPT = ((V + L * NS - 1) // (L * NS)) * L
    V8 = VPT * NS
    FPT = ((F + CHUNK * NS - 1) // (CHUNK * NS)) * CHUNK
    FP = FPT * NS
    NCH = FPT // CHUNK

    f32 = jnp.float32
    pv8 = jnp.pad(pred_verts, ((0, 0), (0, V8 - V), (0, 5))).reshape(B * V8, 8)
    gv8 = jnp.pad(gt_verts, ((0, 0), (0, V8 - V), (0, 5))).reshape(B * V8, 8)
    boff = (jnp.arange(B, dtype=jnp.int32) * V8)[:, None, None]

    def blocks(faces, off):
        fp = jnp.pad(faces.astype(jnp.int32), ((0, 0), (0, FP - F), (0, 0)),
                     constant_values=V) + off
        return jnp.transpose(fp, (0, 2, 1)).reshape(B, 3, NS, NCH, CHUNK)

    pfg = blocks(pred_faces, boff)
    pfs = blocks(pred_faces, 0)
    gfg = blocks(gt_faces, boff)
    zero_rows = jnp.zeros((VPT, 8), f32)

    mesh = plsc.VectorSubcoreMesh(core_axis_name="c", subcore_axis_name="s",
                                  num_cores=NC, num_subcores=NS)
    body = functools.partial(_sc_kernel_body, V, V8, NCH, VPT)
    out = pl.kernel(
        body,
        out_type=jax.ShapeDtypeStruct((NC, NS, 2, L), f32),
        mesh=mesh,
        compiler_params=pltpu.CompilerParams(
            needs_layout_passes=False, use_tc_tiling_on_sc=False),
        scratch_types=[
            pltpu.VMEM_SHARED((V8, 8), f32),
            pltpu.VMEM((3 * NCH, CHUNK), jnp.int32),
            pltpu.VMEM((3 * NCH, CHUNK), jnp.int32),
            pltpu.VMEM((3 * NCH, CHUNK), jnp.int32),
            pltpu.VMEM((CHUNK, 8), f32),
            pltpu.VMEM((CHUNK, 8), f32),
            pltpu.VMEM((CHUNK, 8), f32),
            pltpu.VMEM((CHUNK, 8), f32),
            pltpu.VMEM((CHUNK, 8), f32),
            pltpu.VMEM((CHUNK, 8), f32),
            pltpu.VMEM((CHUNK, 8), f32),
            pltpu.VMEM((VPT // 2, 8), f32),
            pltpu.VMEM((VPT // 2, 8), f32),
            pltpu.VMEM((2, L), f32),
            pltpu.SemaphoreType.DMA,
            pltpu.SemaphoreType.DMA,
        ],
    )(pv8, gv8, pfg, pfs, gfg, zero_rows)

    lap_sum = jnp.sum(out[:, :, 0, :])
    grad_sum = jnp.sum(out[:, :, 1, :])
    return 0.1 * lap_sum / (B * V) + 0.05 * grad_sum / (B * F * 3)
